# baseline (speedup 1.0000x reference)
import functools

import jax
import jax.numpy as jnp
from jax import lax
from jax.experimental import pallas as pl
from jax.experimental.pallas import tpu as pltpu
from jax.experimental.pallas import tpu_sc as plsc

N = 50000
E = 800000
DIN = 128
DH = 256
DOUT = 128
G = 16

NP = 50176
NPC = 25088
NTILES = 16
EPT = E // NTILES
K = 128
TRASH = 32


def _sc_agg(dup, feat, src, dst, invd, BPC, K, RR, FCH, NSTG=2, CH=2000):
  compute_deg = invd is None
  NB = NPC // BPC
  NCH = EPT // CH
  VC = CH // 16
  ROWS = NB // NTILES
  NFC = ROWS // FCH
  FR = dup * FCH
  sh = dup - 1
  EPG = K // dup
  RS = RR * EPG
  GD = NSTG - 1

  out_type = [jax.ShapeDtypeStruct((dup * NP, 128), jnp.float32)]
  if compute_deg:
    out_type.append(jax.ShapeDtypeStruct((NP,), jnp.float32))

  scratch = [
      [pltpu.VMEM((CH,), jnp.int32) for _ in range(2)],
      [pltpu.VMEM((CH,), jnp.int32) for _ in range(2)],
      pltpu.VMEM((RS,), jnp.int32),
      pltpu.VMEM((RS,), jnp.int32),
      pltpu.VMEM((NSTG * dup, EPG), jnp.int32),
      pltpu.VMEM((NSTG * dup, EPG), jnp.int32),
      pltpu.VMEM((NSTG * dup, EPG, 128), jnp.float32),
      pltpu.VMEM((FR, 128), jnp.float32),
      pltpu.VMEM((FCH,), jnp.float32),
      pltpu.VMEM_SHARED((dup * (NB + TRASH), 128), jnp.float32),
      pltpu.SemaphoreType.DMA,
      pltpu.SemaphoreType.DMA,
      pltpu.SemaphoreType.DMA,
  ]
  if compute_deg:
    scratch += [
        pltpu.VMEM((K,), jnp.float32),
        pltpu.VMEM((FCH,), jnp.float32),
        pltpu.VMEM_SHARED((NB + TRASH,), jnp.float32),
    ]

  mesh = plsc.VectorSubcoreMesh(core_axis_name="c", subcore_axis_name="s")

  def body(*refs):
    if compute_deg:
      (feat_h, src_h, dst_h, out_h, invd_h,
       dchunks, schunks, srcbuf, dstbuf, sidxs, didxs, stages, fbuf, invbuf,
       acc, sem, sem2, sem3, ones_s, cbuf, acc1) = refs
    else:
      (feat_h, src_h, dst_h, invd_in_h, out_h,
       dchunks, schunks, srcbuf, dstbuf, sidxs, didxs, stages, fbuf, invbuf,
       acc, sem, sem2, sem3) = refs

    t = lax.axis_index("s")
    c = lax.axis_index("c")
    iota = lax.iota(jnp.int32, 16)
    ebase = t * EPT

    if compute_deg:
      for q in range(K // 16):
        ones_s[pl.ds(q * 16, 16)] = jnp.full((16,), 1.0, jnp.float32)

    def zero_fbuf():
      def zrow(i, _):
        for q in range(8):
          fbuf[i, pl.ds(q * 16, 16)] = jnp.zeros((16,), jnp.float32)
        return 0
      lax.fori_loop(0, FR, zrow, 0)

    def chunk_load(e, ce):
      pltpu.async_copy(dst_h.at[pl.ds(ebase + e * CH, CH)], dchunks[ce], sem2)
      pltpu.async_copy(src_h.at[pl.ds(ebase + e * CH, CH)], schunks[ce], sem2)

    def chunk_wait(e, ce):
      pltpu.make_async_copy(
          dst_h.at[pl.ds(ebase + e * CH, CH)], dchunks[ce], sem2).wait()
      pltpu.make_async_copy(
          src_h.at[pl.ds(ebase + e * CH, CH)], schunks[ce], sem2).wait()

    def bmod(g):
      return (g & 1) if NSTG == 2 else lax.rem(g, NSTG)

    def scat_wait(g):
      so = bmod(g)
      for h in range(dup):
        bb = so * dup + h
        pltpu.make_async_copy(
            stages.at[bb], acc.at[didxs.at[bb]], sem3).wait()
      if compute_deg:
        pltpu.make_async_copy(ones_s, acc1.at[didxs.at[so * dup]], sem3).wait()

    def drain_issue(g):
      so = bmod(g)
      for h in range(dup):
        bb = so * dup + h
        pltpu.make_async_copy(
            feat_h.at[sidxs.at[bb]], stages.at[bb], sem).wait()
        pltpu.async_copy(stages.at[bb], acc.at[didxs.at[bb]], sem3, add=True)
      if compute_deg:
        pltpu.async_copy(ones_s, acc1.at[didxs.at[so * dup]], sem3, add=True)

    def block_body(b, _):
      lo = c * NPC + b * NB

      zero_fbuf()
      def zc(i, _):
        pltpu.sync_copy(fbuf, acc.at[pl.ds(dup * (t * ROWS + i * FCH), FR)])
        return 0
      lax.fori_loop(0, NFC, zc, 0)
      if compute_deg:
        for q in range(FCH // 16):
          invbuf[pl.ds(q * 16, 16)] = jnp.zeros((16,), jnp.float32)
        def zc1(i, _):
          pltpu.sync_copy(invbuf, acc1.at[pl.ds(t * ROWS + i * FCH, FCH)])
          return 0
        lax.fori_loop(0, NFC, zc1, 0)
      plsc.subcore_barrier()

      def fire(f):
        si = bmod(f)
        @pl.when(f >= NSTG)
        def _():
          scat_wait(f - NSTG)
        base = (f & (RR - 1)) * EPG
        for h in range(dup):
          bb = si * dup + h
          for q in range(EPG // 16):
            sv = srcbuf[pl.ds(base + q * 16, 16)]
            sidxs[bb, pl.ds(q * 16, 16)] = sv * dup + h
          pltpu.async_copy(feat_h.at[sidxs.at[bb]], stages.at[bb], sem)
        for h in range(dup):
          bb = si * dup + h
          for q in range(EPG // 16):
            dv = dstbuf[pl.ds(base + q * 16, 16)]
            didxs[bb, pl.ds(q * 16, 16)] = dv * dup + h
        @pl.when(f >= GD)
        def _():
          drain_issue(f - GD)
        return f + 1

      def vreg_body(i, cnt, ce):
        d = dchunks[ce][pl.ds(i * 16, 16)]
        s = schunks[ce][pl.ds(i * 16, 16)]
        du = d - lo
        m = du.astype(jnp.uint32) < jnp.uint32(NB)
        mi = m.astype(jnp.int32)
        pos = (cnt + plsc.cumsum(mi) - 1) & (RS - 1)
        plsc.store_scatter(srcbuf, [pos], s, mask=m)
        plsc.store_scatter(dstbuf, [pos], du, mask=m)
        return cnt + jnp.sum(mi)

      def scan_chunk(e, ce, carry, prefetch):
        cnt, fired = carry
        chunk_wait(e, ce)
        if prefetch:
          chunk_load(e + 1, 1 - ce)
        cnt = lax.fori_loop(0, VC, lambda i, cc: vreg_body(i, cc, ce), cnt)
        fired = lax.while_loop(lambda f: f * EPG + EPG <= cnt, fire, fired)
        return (cnt, fired)

      def chunk_pair(u, carry):
        e = u * 2
        carry = scan_chunk(e, 0, carry, True)
        carry = scan_chunk(e + 1, 1, carry, True)
        return carry

      chunk_load(0, 0)
      carry = lax.fori_loop(0, NCH // 2, chunk_pair, (0, 0))
      if NCH % 2:
        carry = scan_chunk(NCH - 1, 0, carry, False)
      cnt, fired = carry

      npad = (EPG - (cnt & (EPG - 1))) & (EPG - 1)
      nsteps = (npad + 15) >> 4
      def pad_body(k, _):
        pos = (cnt + k * 16 + iota) & (RS - 1)
        plsc.store_scatter(srcbuf, [pos],
                           (t * 997 + k * 16 + iota) & 16383)
        plsc.store_scatter(dstbuf, [pos],
                           NB + ((t * 16 + iota) & (TRASH - 1)))
        return 0
      lax.fori_loop(0, nsteps, pad_body, 0)
      cntp = cnt + npad
      fired = lax.while_loop(lambda f: f * EPG + EPG <= cntp, fire, fired)
      for k in range(GD, 0, -1):
        @pl.when(fired >= k)
        def _(k=k):
          drain_issue(fired - k)
      for k in range(NSTG, 0, -1):
        @pl.when(fired >= k)
        def _(k=k):
          scat_wait(fired - k)
      plsc.subcore_barrier()

      def flush(i, _):
        sl = t * ROWS + i * FCH
        g = lo + sl
        pltpu.sync_copy(acc.at[pl.ds(dup * sl, FR)], fbuf)
        if compute_deg:
          pltpu.sync_copy(acc1.at[pl.ds(sl, FCH)], cbuf)
          for q in range(FCH // 16):
            dv = cbuf[pl.ds(q * 16, 16)]
            invbuf[pl.ds(q * 16, 16)] = 1.0 / jnp.maximum(dv, 1.0)
          pltpu.sync_copy(invbuf, invd_h.at[pl.ds(g, FCH)])
        else:
          pltpu.sync_copy(invd_in_h.at[pl.ds(g, FCH)], invbuf)
        def rowf(r, _):
          sv = plsc.load_gather(invbuf, [jnp.zeros((16,), jnp.int32) + (r >> sh)])
          for q in range(8):
            fbuf[r, pl.ds(q * 16, 16)] = fbuf[r, pl.ds(q * 16, 16)] * sv
          return 0
        lax.fori_loop(0, FR, rowf, 0)
        pltpu.sync_copy(fbuf, out_h.at[pl.ds(dup * g, FR)])
        return 0
      lax.fori_loop(0, NFC, flush, 0)
      plsc.subcore_barrier()
      return 0

    lax.fori_loop(0, BPC, block_body, 0)

  fn = pl.kernel(
      body, out_type=out_type, mesh=mesh, scratch_types=scratch,
      compiler_params=pltpu.CompilerParams(needs_layout_passes=False))
  if compute_deg:
    return fn(feat, src, dst)
  return fn(feat, src, dst, invd)[0]


RB = 512
NRB = NP // RB


def _pre_body(x_ref, w_ref, b_ref, o_ref):
  o_ref[...] = (jnp.dot(x_ref[...], w_ref[...],
                        preferred_element_type=jnp.float32) + b_ref[...])


def _tc_pre(x, wT, b):
  d_in, d_out = wT.shape
  return pl.pallas_call(
      _pre_body,
      grid=(NRB,),
      in_specs=[
          pl.BlockSpec((RB, d_in), lambda i: (i, 0)),
          pl.BlockSpec((d_in, d_out), lambda i: (0, 0)),
          pl.BlockSpec((1, d_out), lambda i: (0, 0)),
      ],
      out_specs=pl.BlockSpec((RB, d_out), lambda i: (i, 0)),
      out_shape=jax.ShapeDtypeStruct((NP, d_out), jnp.float32),
  )(x, wT, b)


def _post_body(agg_ref, w_ref, pre_ref, o_ref):
  o_ref[...] = jnp.maximum(
      jnp.dot(agg_ref[...], w_ref[...], preferred_element_type=jnp.float32)
      + pre_ref[...], 0.0)


def _tc_post(agg, wT, pre):
  d_in, d_out = wT.shape
  return pl.pallas_call(
      _post_body,
      grid=(NRB,),
      in_specs=[
          pl.BlockSpec((RB, d_in), lambda i: (i, 0)),
          pl.BlockSpec((d_in, d_out), lambda i: (0, 0)),
          pl.BlockSpec((RB, d_out), lambda i: (i, 0)),
      ],
      out_specs=pl.BlockSpec((RB, d_out), lambda i: (i, 0)),
      out_shape=jax.ShapeDtypeStruct((NP, d_out), jnp.float32),
  )(agg, wT, pre)


def _mid_body(agg_ref, w2_ref, pre_ref, wl_ref, wr_ref, b_ref, p_ref, r_ref):
  h2 = jnp.maximum(
      jnp.dot(agg_ref[...], w2_ref[...], preferred_element_type=jnp.float32)
      + pre_ref[...], 0.0)
  p_ref[...] = jnp.dot(h2, wl_ref[...], preferred_element_type=jnp.float32)
  r_ref[...] = (jnp.dot(h2, wr_ref[...], preferred_element_type=jnp.float32)
                + b_ref[...])


def _tc_mid(agg2, w2lT, pre2, w3lT, w3rT, b3):
  return pl.pallas_call(
      _mid_body,
      grid=(NRB,),
      in_specs=[
          pl.BlockSpec((RB, DH), lambda i: (i, 0)),
          pl.BlockSpec((DH, DH), lambda i: (0, 0)),
          pl.BlockSpec((RB, DH), lambda i: (i, 0)),
          pl.BlockSpec((DH, DOUT), lambda i: (0, 0)),
          pl.BlockSpec((DH, DOUT), lambda i: (0, 0)),
          pl.BlockSpec((1, DOUT), lambda i: (0, 0)),
      ],
      out_specs=[
          pl.BlockSpec((RB, DOUT), lambda i: (i, 0)),
          pl.BlockSpec((RB, DOUT), lambda i: (i, 0)),
      ],
      out_shape=[
          jax.ShapeDtypeStruct((NP, DOUT), jnp.float32),
          jax.ShapeDtypeStruct((NP, DOUT), jnp.float32),
      ],
  )(agg2, w2lT, pre2, w3lT, w3rT, b3)


def _pool_body(agg_ref, r_ref, b_ref, g_ref, be_ref, o_ref, sums, cnts):
  i = pl.program_id(0)

  @pl.when(i == 0)
  def _():
    sums[...] = jnp.zeros_like(sums)
    cnts[...] = jnp.zeros_like(cnts)

  h3 = agg_ref[...] + r_ref[...]
  bblk = b_ref[0, 0, :]
  oh = (bblk[None, :] ==
        lax.broadcasted_iota(jnp.int32, (G, RB), 0)).astype(jnp.float32)
  sums[...] += jnp.dot(oh, h3, preferred_element_type=jnp.float32)
  cnts[...] += jnp.sum(oh, axis=1, keepdims=True)

  @pl.when(i == NRB - 1)
  def _():
    pooled = sums[...] / jnp.maximum(cnts[...], 1.0)
    mu = jnp.mean(pooled, axis=-1, keepdims=True)
    var = jnp.mean((pooled - mu) ** 2, axis=-1, keepdims=True)
    normed = (pooled - mu) / jnp.sqrt(var + 1e-5)
    o_ref[...] = normed * g_ref[...] + be_ref[...]


def _tc_pool(agg3, r, batch3, gamma, beta):
  return pl.pallas_call(
      _pool_body,
      grid=(NRB,),
      in_specs=[
          pl.BlockSpec((RB, DOUT), lambda i: (i, 0)),
          pl.BlockSpec((RB, DOUT), lambda i: (i, 0)),
          pl.BlockSpec((1, 1, RB), lambda i: (i, 0, 0)),
          pl.BlockSpec((1, DOUT), lambda i: (0, 0)),
          pl.BlockSpec((1, DOUT), lambda i: (0, 0)),
      ],
      out_specs=pl.BlockSpec((G, DOUT), lambda i: (0, 0)),
      out_shape=jax.ShapeDtypeStruct((G, DOUT), jnp.float32),
      scratch_shapes=[
          pltpu.VMEM((G, DOUT), jnp.float32),
          pltpu.VMEM((G, 1), jnp.float32),
      ],
  )(agg3, r, batch3, gamma, beta)


def kernel(x, edge_index, batch, W1l, W1r, b1, W2l, W2r, b2, W3l, W3r, b3,
           gamma, beta):
  src = edge_index[0]
  dst = edge_index[1]
  xp = jnp.pad(x, ((0, NP - N), (0, 0)))
  batch3 = jnp.pad(batch, (0, NP - N), constant_values=G).reshape(NRB, 1, RB)

  pre1 = _tc_pre(xp, W1r.T, b1.reshape(1, DH))
  agg1, invd = _sc_agg(1, xp, src, dst, None, BPC=7, K=128, RR=32, FCH=112)
  h1 = _tc_post(agg1, W1l.T, pre1)

  pre2 = _tc_pre(h1, W2r.T, b2.reshape(1, DH))
  agg2f = _sc_agg(2, h1.reshape(2 * NP, 128), src, dst, invd,
                  BPC=7, K=128, RR=8, FCH=56, NSTG=3, CH=400)
  agg2 = agg2f.reshape(NP, DH)
  p, r = _tc_mid(agg2, W2l.T, pre2, W3l.T, W3r.T, b3.reshape(1, DOUT))

  agg3 = _sc_agg(1, p, src, dst, invd, BPC=4, K=128, RR=32, FCH=56)

  return _tc_pool(agg3, r, batch3, gamma.reshape(1, DOUT),
                  beta.reshape(1, DOUT))


# speedup vs baseline: 1.0303x; 1.0303x over previous
import functools

import jax
import jax.numpy as jnp
from jax import lax
from jax.experimental import pallas as pl
from jax.experimental.pallas import tpu as pltpu
from jax.experimental.pallas import tpu_sc as plsc

N = 50000
E = 800000
DIN = 128
DH = 256
DOUT = 128
G = 16

NP = 50176
NPC = 25088
NTILES = 16
EPT = E // NTILES
K = 128
TRASH = 32


def _sc_agg(dup, feat, src, dst, invd, BPC, K, RR, FCH):
  compute_deg = invd is None
  NB = NPC // BPC
  CH = 2000
  NCH = EPT // CH
  VC = CH // 16
  ROWS = NB // NTILES
  NFC = ROWS // FCH
  FR = dup * FCH
  sh = dup - 1
  EPG = K // dup
  RS = RR * EPG

  out_type = [jax.ShapeDtypeStruct((dup * NP, 128), jnp.float32)]
  if compute_deg:
    out_type.append(jax.ShapeDtypeStruct((NP,), jnp.float32))

  scratch = [
      [pltpu.VMEM((CH,), jnp.int32) for _ in range(2)],
      [pltpu.VMEM((CH,), jnp.int32) for _ in range(2)],
      pltpu.VMEM((RS,), jnp.int32),
      pltpu.VMEM((RS,), jnp.int32),
      pltpu.VMEM((2 * dup, EPG), jnp.int32),
      pltpu.VMEM((2 * dup, EPG), jnp.int32),
      pltpu.VMEM((2 * dup, EPG, 128), jnp.float32),
      pltpu.VMEM((FR, 128), jnp.float32),
      pltpu.VMEM((FCH,), jnp.float32),
      pltpu.VMEM_SHARED((dup * (NB + TRASH), 128), jnp.float32),
      pltpu.SemaphoreType.DMA,
      pltpu.SemaphoreType.DMA,
      pltpu.SemaphoreType.DMA,
  ]
  if compute_deg:
    scratch += [
        pltpu.VMEM((K,), jnp.float32),
        pltpu.VMEM((FCH,), jnp.float32),
        pltpu.VMEM_SHARED((NB + TRASH,), jnp.float32),
    ]

  mesh = plsc.VectorSubcoreMesh(core_axis_name="c", subcore_axis_name="s")

  def body(*refs):
    if compute_deg:
      (feat_h, src_h, dst_h, out_h, invd_h,
       dchunks, schunks, srcbuf, dstbuf, sidxs, didxs, stages, fbuf, invbuf,
       acc, sem, sem2, sem3, ones_s, cbuf, acc1) = refs
    else:
      (feat_h, src_h, dst_h, invd_in_h, out_h,
       dchunks, schunks, srcbuf, dstbuf, sidxs, didxs, stages, fbuf, invbuf,
       acc, sem, sem2, sem3) = refs

    t = lax.axis_index("s")
    c = lax.axis_index("c")
    iota = lax.iota(jnp.int32, 16)
    ebase = t * EPT

    if compute_deg:
      for q in range(K // 16):
        ones_s[pl.ds(q * 16, 16)] = jnp.full((16,), 1.0, jnp.float32)

    def zero_fbuf():
      def zrow(i, _):
        for q in range(8):
          fbuf[i, pl.ds(q * 16, 16)] = jnp.zeros((16,), jnp.float32)
        return 0
      lax.fori_loop(0, FR, zrow, 0)

    def chunk_load(e, ce):
      pltpu.async_copy(dst_h.at[pl.ds(ebase + e * CH, CH)], dchunks[ce], sem2)
      pltpu.async_copy(src_h.at[pl.ds(ebase + e * CH, CH)], schunks[ce], sem2)

    def chunk_wait(e, ce):
      pltpu.make_async_copy(
          dst_h.at[pl.ds(ebase + e * CH, CH)], dchunks[ce], sem2).wait()
      pltpu.make_async_copy(
          src_h.at[pl.ds(ebase + e * CH, CH)], schunks[ce], sem2).wait()

    def scat_wait(so):
      for h in range(dup):
        bb = so * dup + h
        pltpu.make_async_copy(
            stages.at[bb], acc.at[didxs.at[bb]], sem3).wait()
      if compute_deg:
        pltpu.make_async_copy(ones_s, acc1.at[didxs.at[so * dup]], sem3).wait()

    def drain(f):
      so = (f - 1) & 1
      for h in range(dup):
        bb = so * dup + h
        pltpu.make_async_copy(
            feat_h.at[sidxs.at[bb]], stages.at[bb], sem).wait()
        pltpu.async_copy(stages.at[bb], acc.at[didxs.at[bb]], sem3, add=True)
      if compute_deg:
        pltpu.async_copy(ones_s, acc1.at[didxs.at[so * dup]], sem3, add=True)

    def block_body(b, _):
      lo = c * NPC + b * NB

      zero_fbuf()
      def zc(i, _):
        pltpu.sync_copy(fbuf, acc.at[pl.ds(dup * (t * ROWS + i * FCH), FR)])
        return 0
      lax.fori_loop(0, NFC, zc, 0)
      if compute_deg:
        for qo in ([*range(0, FCH - 15, 16)] + ([FCH - 16] if FCH % 16 else [])):
          invbuf[pl.ds(qo, 16)] = jnp.zeros((16,), jnp.float32)
        def zc1(i, _):
          pltpu.sync_copy(invbuf, acc1.at[pl.ds(t * ROWS + i * FCH, FCH)])
          return 0
        lax.fori_loop(0, NFC, zc1, 0)
      plsc.subcore_barrier()

      def fire(f):
        si = f & 1
        @pl.when(f > 1)
        def _():
          scat_wait(si)
        base = (f & (RR - 1)) * EPG
        for h in range(dup):
          bb = si * dup + h
          for q in range(EPG // 16):
            sv = srcbuf[pl.ds(base + q * 16, 16)]
            sidxs[bb, pl.ds(q * 16, 16)] = sv * dup + h
          pltpu.async_copy(feat_h.at[sidxs.at[bb]], stages.at[bb], sem)
        for h in range(dup):
          bb = si * dup + h
          for q in range(EPG // 16):
            dv = dstbuf[pl.ds(base + q * 16, 16)]
            didxs[bb, pl.ds(q * 16, 16)] = dv * dup + h
        @pl.when(f > 0)
        def _():
          drain(f)
        return f + 1

      def vreg_body(i, cnt, ce):
        d = dchunks[ce][pl.ds(i * 16, 16)]
        s = schunks[ce][pl.ds(i * 16, 16)]
        du = d - lo
        m = du.astype(jnp.uint32) < jnp.uint32(NB)
        mi = m.astype(jnp.int32)
        pos = (cnt + plsc.cumsum(mi) - 1) & (RS - 1)
        plsc.store_scatter(srcbuf, [pos], s, mask=m)
        plsc.store_scatter(dstbuf, [pos], du, mask=m)
        return cnt + jnp.sum(mi)

      def scan_chunk(e, ce, carry, prefetch):
        cnt, fired = carry
        chunk_wait(e, ce)
        if prefetch:
          chunk_load(e + 1, 1 - ce)
        cnt = lax.fori_loop(0, VC, lambda i, cc: vreg_body(i, cc, ce), cnt)
        fired = lax.while_loop(lambda f: f * EPG + EPG <= cnt, fire, fired)
        return (cnt, fired)

      def chunk_pair(u, carry):
        e = u * 2
        carry = scan_chunk(e, 0, carry, True)
        carry = scan_chunk(e + 1, 1, carry, True)
        return carry

      chunk_load(0, 0)
      carry = lax.fori_loop(0, NCH // 2, chunk_pair, (0, 0))
      if NCH % 2:
        carry = scan_chunk(NCH - 1, 0, carry, False)
      cnt, fired = carry

      npad = (EPG - (cnt & (EPG - 1))) & (EPG - 1)
      nsteps = (npad + 15) >> 4
      def pad_body(k, _):
        pos = (cnt + k * 16 + iota) & (RS - 1)
        plsc.store_scatter(srcbuf, [pos],
                           (t * 997 + k * 16 + iota) & 16383)
        plsc.store_scatter(dstbuf, [pos],
                           NB + ((t * 16 + iota) & (TRASH - 1)))
        return 0
      lax.fori_loop(0, nsteps, pad_body, 0)
      cntp = cnt + npad
      fired = lax.while_loop(lambda f: f * EPG + EPG <= cntp, fire, fired)
      @pl.when(fired > 0)
      def _():
        drain(fired)
      @pl.when(fired > 1)
      def _():
        scat_wait((fired - 2) & 1)
      @pl.when(fired > 0)
      def _():
        scat_wait((fired - 1) & 1)
      plsc.subcore_barrier()

      def flush(i, _):
        sl = t * ROWS + i * FCH
        g = lo + sl
        pltpu.sync_copy(acc.at[pl.ds(dup * sl, FR)], fbuf)
        if compute_deg:
          pltpu.sync_copy(acc1.at[pl.ds(sl, FCH)], cbuf)
          for qo in ([*range(0, FCH - 15, 16)] + ([FCH - 16] if FCH % 16 else [])):
            dv = cbuf[pl.ds(qo, 16)]
            invbuf[pl.ds(qo, 16)] = 1.0 / jnp.maximum(dv, 1.0)
          pltpu.sync_copy(invbuf, invd_h.at[pl.ds(g, FCH)])
        else:
          pltpu.sync_copy(invd_in_h.at[pl.ds(g, FCH)], invbuf)
        def rowf(r, _):
          sv = plsc.load_gather(invbuf, [jnp.zeros((16,), jnp.int32) + (r >> sh)])
          for q in range(8):
            fbuf[r, pl.ds(q * 16, 16)] = fbuf[r, pl.ds(q * 16, 16)] * sv
          return 0
        lax.fori_loop(0, FR, rowf, 0)
        pltpu.sync_copy(fbuf, out_h.at[pl.ds(dup * g, FR)])
        return 0
      lax.fori_loop(0, NFC, flush, 0)
      plsc.subcore_barrier()
      return 0

    lax.fori_loop(0, BPC, block_body, 0)

  fn = pl.kernel(
      body, out_type=out_type, mesh=mesh, scratch_types=scratch,
      compiler_params=pltpu.CompilerParams(needs_layout_passes=False))
  if compute_deg:
    return fn(feat, src, dst)
  return fn(feat, src, dst, invd)[0]


RB = 512
NRB = NP // RB


def _pre_body(x_ref, w_ref, b_ref, o_ref):
  o_ref[...] = (jnp.dot(x_ref[...], w_ref[...],
                        preferred_element_type=jnp.float32) + b_ref[...])


def _tc_pre(x, wT, b):
  d_in, d_out = wT.shape
  return pl.pallas_call(
      _pre_body,
      grid=(NRB,),
      in_specs=[
          pl.BlockSpec((RB, d_in), lambda i: (i, 0)),
          pl.BlockSpec((d_in, d_out), lambda i: (0, 0)),
          pl.BlockSpec((1, d_out), lambda i: (0, 0)),
      ],
      out_specs=pl.BlockSpec((RB, d_out), lambda i: (i, 0)),
      out_shape=jax.ShapeDtypeStruct((NP, d_out), jnp.float32),
  )(x, wT, b)


def _post_body(agg_ref, w_ref, pre_ref, o_ref):
  o_ref[...] = jnp.maximum(
      jnp.dot(agg_ref[...], w_ref[...], preferred_element_type=jnp.float32)
      + pre_ref[...], 0.0)


def _tc_post(agg, wT, pre):
  d_in, d_out = wT.shape
  return pl.pallas_call(
      _post_body,
      grid=(NRB,),
      in_specs=[
          pl.BlockSpec((RB, d_in), lambda i: (i, 0)),
          pl.BlockSpec((d_in, d_out), lambda i: (0, 0)),
          pl.BlockSpec((RB, d_out), lambda i: (i, 0)),
      ],
      out_specs=pl.BlockSpec((RB, d_out), lambda i: (i, 0)),
      out_shape=jax.ShapeDtypeStruct((NP, d_out), jnp.float32),
  )(agg, wT, pre)


def _mid_body(agg_ref, w2_ref, pre_ref, wl_ref, wr_ref, b_ref, p_ref, r_ref):
  h2 = jnp.maximum(
      jnp.dot(agg_ref[...], w2_ref[...], preferred_element_type=jnp.float32)
      + pre_ref[...], 0.0)
  p_ref[...] = jnp.dot(h2, wl_ref[...], preferred_element_type=jnp.float32)
  r_ref[...] = (jnp.dot(h2, wr_ref[...], preferred_element_type=jnp.float32)
                + b_ref[...])


def _tc_mid(agg2, w2lT, pre2, w3lT, w3rT, b3):
  return pl.pallas_call(
      _mid_body,
      grid=(NRB,),
      in_specs=[
          pl.BlockSpec((RB, DH), lambda i: (i, 0)),
          pl.BlockSpec((DH, DH), lambda i: (0, 0)),
          pl.BlockSpec((RB, DH), lambda i: (i, 0)),
          pl.BlockSpec((DH, DOUT), lambda i: (0, 0)),
          pl.BlockSpec((DH, DOUT), lambda i: (0, 0)),
          pl.BlockSpec((1, DOUT), lambda i: (0, 0)),
      ],
      out_specs=[
          pl.BlockSpec((RB, DOUT), lambda i: (i, 0)),
          pl.BlockSpec((RB, DOUT), lambda i: (i, 0)),
      ],
      out_shape=[
          jax.ShapeDtypeStruct((NP, DOUT), jnp.float32),
          jax.ShapeDtypeStruct((NP, DOUT), jnp.float32),
      ],
  )(agg2, w2lT, pre2, w3lT, w3rT, b3)


def _pool_body(agg_ref, r_ref, b_ref, g_ref, be_ref, o_ref, sums, cnts):
  i = pl.program_id(0)

  @pl.when(i == 0)
  def _():
    sums[...] = jnp.zeros_like(sums)
    cnts[...] = jnp.zeros_like(cnts)

  h3 = agg_ref[...] + r_ref[...]
  bblk = b_ref[0, 0, :]
  oh = (bblk[None, :] ==
        lax.broadcasted_iota(jnp.int32, (G, RB), 0)).astype(jnp.float32)
  sums[...] += jnp.dot(oh, h3, preferred_element_type=jnp.float32)
  cnts[...] += jnp.sum(oh, axis=1, keepdims=True)

  @pl.when(i == NRB - 1)
  def _():
    pooled = sums[...] / jnp.maximum(cnts[...], 1.0)
    mu = jnp.mean(pooled, axis=-1, keepdims=True)
    var = jnp.mean((pooled - mu) ** 2, axis=-1, keepdims=True)
    normed = (pooled - mu) / jnp.sqrt(var + 1e-5)
    o_ref[...] = normed * g_ref[...] + be_ref[...]


def _tc_pool(agg3, r, batch3, gamma, beta):
  return pl.pallas_call(
      _pool_body,
      grid=(NRB,),
      in_specs=[
          pl.BlockSpec((RB, DOUT), lambda i: (i, 0)),
          pl.BlockSpec((RB, DOUT), lambda i: (i, 0)),
          pl.BlockSpec((1, 1, RB), lambda i: (i, 0, 0)),
          pl.BlockSpec((1, DOUT), lambda i: (0, 0)),
          pl.BlockSpec((1, DOUT), lambda i: (0, 0)),
      ],
      out_specs=pl.BlockSpec((G, DOUT), lambda i: (0, 0)),
      out_shape=jax.ShapeDtypeStruct((G, DOUT), jnp.float32),
      scratch_shapes=[
          pltpu.VMEM((G, DOUT), jnp.float32),
          pltpu.VMEM((G, 1), jnp.float32),
      ],
  )(agg3, r, batch3, gamma, beta)


def kernel(x, edge_index, batch, W1l, W1r, b1, W2l, W2r, b2, W3l, W3r, b3,
           gamma, beta):
  src = edge_index[0]
  dst = edge_index[1]
  xp = jnp.pad(x, ((0, NP - N), (0, 0)))
  batch3 = jnp.pad(batch, (0, NP - N), constant_values=G).reshape(NRB, 1, RB)

  pre1 = _tc_pre(xp, W1r.T, b1.reshape(1, DH))
  agg1, invd = _sc_agg(1, xp, src, dst, None, BPC=4, K=128, RR=32, FCH=56)
  h1 = _tc_post(agg1, W1l.T, pre1)

  pre2 = _tc_pre(h1, W2r.T, b2.reshape(1, DH))
  agg2f = _sc_agg(2, h1.reshape(2 * NP, 128), src, dst, invd,
                  BPC=7, K=128, RR=64, FCH=56)
  agg2 = agg2f.reshape(NP, DH)
  p, r = _tc_mid(agg2, W2l.T, pre2, W3l.T, W3r.T, b3.reshape(1, DOUT))

  agg3 = _sc_agg(1, p, src, dst, invd, BPC=4, K=128, RR=32, FCH=56)

  return _tc_pool(agg3, r, batch3, gamma.reshape(1, DOUT),
                  beta.reshape(1, DOUT))


# speedup vs baseline: 1.0308x; 1.0005x over previous
import jax
import jax.numpy as jnp
from jax import lax
from jax.experimental import pallas as pl
from jax.experimental.pallas import tpu as pltpu
from jax.experimental.pallas import tpu_sc as plsc

N = 50000
E = 800000
DIN = 128
DH = 256
DOUT = 128
G = 16

NP = 50176
NPC = 25088
NTILES = 16
EPT = E // NTILES
TRASH = 32


def _sc_agg(dup, feat, src, dst, invd, BPC, K, RR, FCH):
  compute_deg = invd is None
  NB = NPC // BPC
  CH = 2000
  NCH = EPT // CH
  VC = CH // 16
  ROWS = NB // NTILES
  NFC = ROWS // FCH
  FR = dup * FCH
  sh = dup - 1
  EPG = K // dup
  RS = RR * EPG

  out_type = [jax.ShapeDtypeStruct((dup * NP, 128), jnp.float32)]
  if compute_deg:
    out_type.append(jax.ShapeDtypeStruct((NP,), jnp.float32))

  scratch = [
      [pltpu.VMEM((CH,), jnp.int32) for _ in range(2)],
      [pltpu.VMEM((CH,), jnp.int32) for _ in range(2)],
      pltpu.VMEM((RS,), jnp.int32),
      pltpu.VMEM((RS,), jnp.int32),
      pltpu.VMEM((2 * dup, EPG), jnp.int32),
      pltpu.VMEM((2 * dup, EPG), jnp.int32),
      pltpu.VMEM((2 * dup, EPG, 128), jnp.float32),
      pltpu.VMEM((FR, 128), jnp.float32),
      pltpu.VMEM((FCH,), jnp.float32),
      pltpu.VMEM_SHARED((dup * (NB + TRASH), 128), jnp.float32),
      pltpu.SemaphoreType.DMA,
      pltpu.SemaphoreType.DMA,
      pltpu.SemaphoreType.DMA,
  ]
  if compute_deg:
    scratch += [
        pltpu.VMEM((K,), jnp.float32),
        pltpu.VMEM((FCH,), jnp.float32),
        pltpu.VMEM_SHARED((NB + TRASH,), jnp.float32),
    ]

  mesh = plsc.VectorSubcoreMesh(core_axis_name="c", subcore_axis_name="s")

  def body(*refs):
    if compute_deg:
      (feat_h, src_h, dst_h, out_h, invd_h,
       dchunks, schunks, srcbuf, dstbuf, sidxs, didxs, stages, fbuf, invbuf,
       acc, sem, sem2, sem3, ones_s, cbuf, acc1) = refs
    else:
      (feat_h, src_h, dst_h, invd_in_h, out_h,
       dchunks, schunks, srcbuf, dstbuf, sidxs, didxs, stages, fbuf, invbuf,
       acc, sem, sem2, sem3) = refs

    t = lax.axis_index("s")
    c = lax.axis_index("c")
    iota = lax.iota(jnp.int32, 16)
    ebase = t * EPT

    if compute_deg:
      for q in range(K // 16):
        ones_s[pl.ds(q * 16, 16)] = jnp.full((16,), 1.0, jnp.float32)

    def zero_fbuf():
      def zrow(i, _):
        for q in range(8):
          fbuf[i, pl.ds(q * 16, 16)] = jnp.zeros((16,), jnp.float32)
        return 0
      lax.fori_loop(0, FR, zrow, 0)

    def chunk_load(e, ce):
      pltpu.async_copy(dst_h.at[pl.ds(ebase + e * CH, CH)], dchunks[ce], sem2)
      pltpu.async_copy(src_h.at[pl.ds(ebase + e * CH, CH)], schunks[ce], sem2)

    def chunk_wait(e, ce):
      pltpu.make_async_copy(
          dst_h.at[pl.ds(ebase + e * CH, CH)], dchunks[ce], sem2).wait()
      pltpu.make_async_copy(
          src_h.at[pl.ds(ebase + e * CH, CH)], schunks[ce], sem2).wait()

    def scat_wait(so):
      for h in range(dup):
        bb = so * dup + h
        pltpu.make_async_copy(
            stages.at[bb], acc.at[didxs.at[bb]], sem3).wait()
      if compute_deg:
        pltpu.make_async_copy(ones_s, acc1.at[didxs.at[so * dup]], sem3).wait()

    def drain(f):
      so = (f - 1) & 1
      for h in range(dup):
        bb = so * dup + h
        pltpu.make_async_copy(
            feat_h.at[sidxs.at[bb]], stages.at[bb], sem).wait()
        pltpu.async_copy(stages.at[bb], acc.at[didxs.at[bb]], sem3, add=True)
      if compute_deg:
        pltpu.async_copy(ones_s, acc1.at[didxs.at[so * dup]], sem3, add=True)

    def block_body(b, _):
      lo = c * NPC + b * NB

      zero_fbuf()
      def zc(i, _):
        pltpu.sync_copy(fbuf, acc.at[pl.ds(dup * (t * ROWS + i * FCH), FR)])
        return 0
      lax.fori_loop(0, NFC, zc, 0)
      if compute_deg:
        for qo in ([*range(0, FCH - 15, 16)] + ([FCH - 16] if FCH % 16 else [])):
          invbuf[pl.ds(qo, 16)] = jnp.zeros((16,), jnp.float32)
        def zc1(i, _):
          pltpu.sync_copy(invbuf, acc1.at[pl.ds(t * ROWS + i * FCH, FCH)])
          return 0
        lax.fori_loop(0, NFC, zc1, 0)
      plsc.subcore_barrier()

      def fire(f):
        si = f & 1
        @pl.when(f > 1)
        def _():
          scat_wait(si)
        base = (f & (RR - 1)) * EPG
        for h in range(dup):
          bb = si * dup + h
          for q in range(EPG // 16):
            sv = srcbuf[pl.ds(base + q * 16, 16)]
            sidxs[bb, pl.ds(q * 16, 16)] = sv * dup + h
          pltpu.async_copy(feat_h.at[sidxs.at[bb]], stages.at[bb], sem)
        for h in range(dup):
          bb = si * dup + h
          for q in range(EPG // 16):
            dv = dstbuf[pl.ds(base + q * 16, 16)]
            didxs[bb, pl.ds(q * 16, 16)] = dv * dup + h
        @pl.when(f > 0)
        def _():
          drain(f)
        return f + 1

      def vreg_body(i, cnt, ce):
        d = dchunks[ce][pl.ds(i * 16, 16)]
        s = schunks[ce][pl.ds(i * 16, 16)]
        du = d - lo
        m = du.astype(jnp.uint32) < jnp.uint32(NB)
        mi = m.astype(jnp.int32)
        pos = (cnt + plsc.cumsum(mi) - 1) & (RS - 1)
        plsc.store_scatter(srcbuf, [pos], s, mask=m)
        plsc.store_scatter(dstbuf, [pos], du, mask=m)
        return cnt + jnp.sum(mi)

      def scan_chunk(e, ce, carry, prefetch):
        cnt, fired = carry
        chunk_wait(e, ce)
        if prefetch:
          chunk_load(e + 1, 1 - ce)
        cnt = lax.fori_loop(0, VC, lambda i, cc: vreg_body(i, cc, ce), cnt)
        fired = lax.while_loop(lambda f: f * EPG + EPG <= cnt, fire, fired)
        return (cnt, fired)

      def chunk_pair(u, carry):
        e = u * 2
        carry = scan_chunk(e, 0, carry, True)
        carry = scan_chunk(e + 1, 1, carry, True)
        return carry

      chunk_load(0, 0)
      carry = lax.fori_loop(0, NCH // 2, chunk_pair, (0, 0))
      if NCH % 2:
        carry = scan_chunk(NCH - 1, 0, carry, False)
      cnt, fired = carry

      npad = (EPG - (cnt & (EPG - 1))) & (EPG - 1)
      nsteps = (npad + 15) >> 4
      def pad_body(k, _):
        pos = (cnt + k * 16 + iota) & (RS - 1)
        plsc.store_scatter(srcbuf, [pos],
                           (t * 997 + k * 16 + iota) & 16383)
        plsc.store_scatter(dstbuf, [pos],
                           NB + ((t * 16 + iota) & (TRASH - 1)))
        return 0
      lax.fori_loop(0, nsteps, pad_body, 0)
      cntp = cnt + npad
      fired = lax.while_loop(lambda f: f * EPG + EPG <= cntp, fire, fired)
      @pl.when(fired > 0)
      def _():
        drain(fired)
      @pl.when(fired > 1)
      def _():
        scat_wait((fired - 2) & 1)
      @pl.when(fired > 0)
      def _():
        scat_wait((fired - 1) & 1)
      plsc.subcore_barrier()

      def flush(i, _):
        sl = t * ROWS + i * FCH
        g = lo + sl
        pltpu.sync_copy(acc.at[pl.ds(dup * sl, FR)], fbuf)
        if compute_deg:
          pltpu.sync_copy(acc1.at[pl.ds(sl, FCH)], cbuf)
          for qo in ([*range(0, FCH - 15, 16)] + ([FCH - 16] if FCH % 16 else [])):
            dv = cbuf[pl.ds(qo, 16)]
            invbuf[pl.ds(qo, 16)] = 1.0 / jnp.maximum(dv, 1.0)
          pltpu.sync_copy(invbuf, invd_h.at[pl.ds(g, FCH)])
        else:
          pltpu.sync_copy(invd_in_h.at[pl.ds(g, FCH)], invbuf)
        def rowf(r, _):
          sv = plsc.load_gather(invbuf, [jnp.zeros((16,), jnp.int32) + (r >> sh)])
          for q in range(8):
            fbuf[r, pl.ds(q * 16, 16)] = fbuf[r, pl.ds(q * 16, 16)] * sv
          return 0
        lax.fori_loop(0, FR, rowf, 0)
        pltpu.sync_copy(fbuf, out_h.at[pl.ds(dup * g, FR)])
        return 0
      lax.fori_loop(0, NFC, flush, 0)
      plsc.subcore_barrier()
      return 0

    lax.fori_loop(0, BPC, block_body, 0)

  fn = pl.kernel(
      body, out_type=out_type, mesh=mesh, scratch_types=scratch,
      compiler_params=pltpu.CompilerParams(needs_layout_passes=False))
  if compute_deg:
    return fn(feat, src, dst)
  return fn(feat, src, dst, invd)[0]


RB = 512
NRB = NP // RB


def _pre_body(x_ref, w_ref, b_ref, o_ref):
  o_ref[...] = (jnp.dot(x_ref[...], w_ref[...],
                        preferred_element_type=jnp.float32) + b_ref[...])


def _tc_pre(x, wT, b):
  d_in, d_out = wT.shape
  return pl.pallas_call(
      _pre_body,
      grid=(NRB,),
      in_specs=[
          pl.BlockSpec((RB, d_in), lambda i: (i, 0)),
          pl.BlockSpec((d_in, d_out), lambda i: (0, 0)),
          pl.BlockSpec((1, d_out), lambda i: (0, 0)),
      ],
      out_specs=pl.BlockSpec((RB, d_out), lambda i: (i, 0)),
      out_shape=jax.ShapeDtypeStruct((NP, d_out), jnp.float32),
  )(x, wT, b)


def _post_body(agg_ref, w_ref, pre_ref, o_ref):
  o_ref[...] = jnp.maximum(
      jnp.dot(agg_ref[...], w_ref[...], preferred_element_type=jnp.float32)
      + pre_ref[...], 0.0)


def _tc_post(agg, wT, pre):
  d_in, d_out = wT.shape
  return pl.pallas_call(
      _post_body,
      grid=(NRB,),
      in_specs=[
          pl.BlockSpec((RB, d_in), lambda i: (i, 0)),
          pl.BlockSpec((d_in, d_out), lambda i: (0, 0)),
          pl.BlockSpec((RB, d_out), lambda i: (i, 0)),
      ],
      out_specs=pl.BlockSpec((RB, d_out), lambda i: (i, 0)),
      out_shape=jax.ShapeDtypeStruct((NP, d_out), jnp.float32),
  )(agg, wT, pre)


def _mid_body(agg_ref, w2_ref, pre_ref, wl_ref, wr_ref, b_ref, p_ref, r_ref):
  h2 = jnp.maximum(
      jnp.dot(agg_ref[...], w2_ref[...], preferred_element_type=jnp.float32)
      + pre_ref[...], 0.0)
  p_ref[...] = jnp.dot(h2, wl_ref[...], preferred_element_type=jnp.float32)
  r_ref[...] = (jnp.dot(h2, wr_ref[...], preferred_element_type=jnp.float32)
                + b_ref[...])


def _tc_mid(agg2, w2lT, pre2, w3lT, w3rT, b3):
  return pl.pallas_call(
      _mid_body,
      grid=(NRB,),
      in_specs=[
          pl.BlockSpec((RB, DH), lambda i: (i, 0)),
          pl.BlockSpec((DH, DH), lambda i: (0, 0)),
          pl.BlockSpec((RB, DH), lambda i: (i, 0)),
          pl.BlockSpec((DH, DOUT), lambda i: (0, 0)),
          pl.BlockSpec((DH, DOUT), lambda i: (0, 0)),
          pl.BlockSpec((1, DOUT), lambda i: (0, 0)),
      ],
      out_specs=[
          pl.BlockSpec((RB, DOUT), lambda i: (i, 0)),
          pl.BlockSpec((RB, DOUT), lambda i: (i, 0)),
      ],
      out_shape=[
          jax.ShapeDtypeStruct((NP, DOUT), jnp.float32),
          jax.ShapeDtypeStruct((NP, DOUT), jnp.float32),
      ],
  )(agg2, w2lT, pre2, w3lT, w3rT, b3)


def _pool_body(agg_ref, r_ref, b_ref, g_ref, be_ref, o_ref, sums, cnts):
  i = pl.program_id(0)

  @pl.when(i == 0)
  def _():
    sums[...] = jnp.zeros_like(sums)
    cnts[...] = jnp.zeros_like(cnts)

  h3 = agg_ref[...] + r_ref[...]
  bblk = b_ref[0, 0, :]
  oh = (bblk[None, :] ==
        lax.broadcasted_iota(jnp.int32, (G, RB), 0)).astype(jnp.float32)
  sums[...] += jnp.dot(oh, h3, preferred_element_type=jnp.float32)
  cnts[...] += jnp.sum(oh, axis=1, keepdims=True)

  @pl.when(i == NRB - 1)
  def _():
    pooled = sums[...] / jnp.maximum(cnts[...], 1.0)
    mu = jnp.mean(pooled, axis=-1, keepdims=True)
    var = jnp.mean((pooled - mu) ** 2, axis=-1, keepdims=True)
    normed = (pooled - mu) / jnp.sqrt(var + 1e-5)
    o_ref[...] = normed * g_ref[...] + be_ref[...]


def _tc_pool(agg3, r, batch3, gamma, beta):
  return pl.pallas_call(
      _pool_body,
      grid=(NRB,),
      in_specs=[
          pl.BlockSpec((RB, DOUT), lambda i: (i, 0)),
          pl.BlockSpec((RB, DOUT), lambda i: (i, 0)),
          pl.BlockSpec((1, 1, RB), lambda i: (i, 0, 0)),
          pl.BlockSpec((1, DOUT), lambda i: (0, 0)),
          pl.BlockSpec((1, DOUT), lambda i: (0, 0)),
      ],
      out_specs=pl.BlockSpec((G, DOUT), lambda i: (0, 0)),
      out_shape=jax.ShapeDtypeStruct((G, DOUT), jnp.float32),
      scratch_shapes=[
          pltpu.VMEM((G, DOUT), jnp.float32),
          pltpu.VMEM((G, 1), jnp.float32),
      ],
  )(agg3, r, batch3, gamma, beta)


def kernel(x, edge_index, batch, W1l, W1r, b1, W2l, W2r, b2, W3l, W3r, b3,
           gamma, beta):
  src = edge_index[0]
  dst = edge_index[1]
  xp = jnp.pad(x, ((0, NP - N), (0, 0)))
  batch3 = jnp.pad(batch, (0, NP - N), constant_values=G).reshape(NRB, 1, RB)

  pre1 = _tc_pre(xp, W1r.T, b1.reshape(1, DH))
  agg1, invd = _sc_agg(1, xp, src, dst, None, BPC=4, K=128, RR=32, FCH=56)
  h1 = _tc_post(agg1, W1l.T, pre1)

  pre2 = _tc_pre(h1, W2r.T, b2.reshape(1, DH))
  agg2f = _sc_agg(2, h1.reshape(2 * NP, 128), src, dst, invd,
                  BPC=7, K=128, RR=64, FCH=56)
  agg2 = agg2f.reshape(NP, DH)
  p, r = _tc_mid(agg2, W2l.T, pre2, W3l.T, W3r.T, b3.reshape(1, DOUT))

  agg3 = _sc_agg(1, p, src, dst, invd, BPC=4, K=128, RR=32, FCH=56)

  return _tc_pool(agg3, r, batch3, gamma.reshape(1, DOUT),
                  beta.reshape(1, DOUT))
